# baseline (speedup 1.0000x reference)
import functools

import jax
import jax.numpy as jnp
from jax.experimental import pallas as pl

D = 4096
H = D // 2
E = 8
TEMPERATURE = 1.0
MIN_USAGE_THRESHOLD = 0.01

TM = 512


def _router_block(x_ref, w1t_ref, b1_ref, w2t_ref, b2_ref, thr_ref, uc_ref,
                  ss_ref, out_ref, mask_ref):
    xb = x_ref[...]
    h = jax.lax.dot(xb, w1t_ref[...],
                    preferred_element_type=jnp.float32)
    h = jnp.maximum(h + b1_ref[...], 0.0)
    logits = jax.lax.dot(h.astype(jnp.bfloat16), w2t_ref[...],
                         preferred_element_type=jnp.float32)
    logits = (logits + b2_ref[...]) / TEMPERATURE

    uc = uc_ref[...]
    ur = uc / jnp.sum(uc)
    ss = ss_ref[...]
    sp = jnp.exp(ss - jnp.max(ss))
    sp = sp / jnp.sum(sp)
    maskf = jnp.where((ur > MIN_USAGE_THRESHOLD) & (sp > 0.05), 1.0, 0.0)
    mask_ref[...] = maskf

    logits = jnp.where(maskf > 0.5, logits, -jnp.inf)
    m = jnp.max(logits, axis=-1, keepdims=True)
    e = jnp.exp(logits - m)
    p = e / jnp.sum(e, axis=-1, keepdims=True)
    p = jnp.where(p > thr_ref[...], p, 0.0)

    iota = jax.lax.broadcasted_iota(jnp.int32, p.shape, 1)
    m1 = jnp.max(p, axis=-1, keepdims=True)
    i1 = jnp.min(jnp.where(p == m1, iota, E), axis=-1, keepdims=True)
    p2 = jnp.where(iota == i1, -1.0, p)
    m2 = jnp.max(p2, axis=-1, keepdims=True)
    i2 = jnp.min(jnp.where(p2 == m2, iota, E), axis=-1, keepdims=True)
    denom = m1 + m2 + 1e-9
    sel = (iota == i1) | (iota == i2)
    out_ref[...] = jnp.where(sel, p / denom, 0.0)


@functools.partial(jax.jit, static_argnames=())
def kernel(inputs, W1, b1, W2, b2, routing_thresholds, usage_counts,
           specialization_scores):
    B, S, d = inputs.shape
    n = B * S
    x = inputs.reshape(n, d).astype(jnp.bfloat16)
    w1t = W1.T.astype(jnp.bfloat16)
    w2t = W2.T.astype(jnp.bfloat16)
    grid = (n // TM,)

    out, maskf = pl.pallas_call(
        _router_block,
        grid=grid,
        in_specs=[
            pl.BlockSpec((TM, D), lambda i: (i, 0)),
            pl.BlockSpec((D, H), lambda i: (0, 0)),
            pl.BlockSpec((1, H), lambda i: (0, 0)),
            pl.BlockSpec((H, E), lambda i: (0, 0)),
            pl.BlockSpec((1, E), lambda i: (0, 0)),
            pl.BlockSpec((1, E), lambda i: (0, 0)),
            pl.BlockSpec((1, E), lambda i: (0, 0)),
            pl.BlockSpec((1, E), lambda i: (0, 0)),
        ],
        out_specs=[
            pl.BlockSpec((TM, E), lambda i: (i, 0)),
            pl.BlockSpec((1, E), lambda i: (0, 0)),
        ],
        out_shape=[
            jax.ShapeDtypeStruct((n, E), jnp.float32),
            jax.ShapeDtypeStruct((1, E), jnp.float32),
        ],
    )(x, w1t, b1.reshape(1, H), w2t, b2.reshape(1, E),
      routing_thresholds.reshape(1, E), usage_counts.reshape(1, E),
      specialization_scores.reshape(1, E))

    routing_weights = out.reshape(B, S, E)
    expert_mask = maskf.reshape(E) > 0.5
    return (routing_weights, expert_mask)


# speedup vs baseline: 1.3248x; 1.3248x over previous
import functools

import jax
import jax.numpy as jnp
from jax.experimental import pallas as pl

D = 4096
H = D // 2
E = 8
TEMPERATURE = 1.0
MIN_USAGE_THRESHOLD = 0.01

TM = 512


def _router_block(x_ref, w1t_ref, b1_ref, w2t_ref, b2_ref, thr_ref, uc_ref,
                  ss_ref, out_ref, mask_ref):
    xb = x_ref[...].astype(jnp.bfloat16)
    h = jax.lax.dot(xb, w1t_ref[...],
                    preferred_element_type=jnp.float32)
    h = jnp.maximum(h + b1_ref[...], 0.0)
    logits = jax.lax.dot(h.astype(jnp.bfloat16), w2t_ref[...],
                         preferred_element_type=jnp.float32)
    logits = (logits + b2_ref[...]) / TEMPERATURE

    uc = uc_ref[...]
    ur = uc / jnp.sum(uc)
    ss = ss_ref[...]
    sp = jnp.exp(ss - jnp.max(ss))
    sp = sp / jnp.sum(sp)
    maskf = jnp.where((ur > MIN_USAGE_THRESHOLD) & (sp > 0.05), 1.0, 0.0)
    mask_ref[...] = maskf

    logits = jnp.where(maskf > 0.5, logits, -jnp.inf)
    m = jnp.max(logits, axis=-1, keepdims=True)
    e = jnp.exp(logits - m)
    p = e / jnp.sum(e, axis=-1, keepdims=True)
    p = jnp.where(p > thr_ref[...], p, 0.0)

    iota = jax.lax.broadcasted_iota(jnp.int32, p.shape, 1)
    m1 = jnp.max(p, axis=-1, keepdims=True)
    i1 = jnp.min(jnp.where(p == m1, iota, E), axis=-1, keepdims=True)
    p2 = jnp.where(iota == i1, -1.0, p)
    m2 = jnp.max(p2, axis=-1, keepdims=True)
    i2 = jnp.min(jnp.where(p2 == m2, iota, E), axis=-1, keepdims=True)
    denom = m1 + m2 + 1e-9
    sel = (iota == i1) | (iota == i2)
    out_ref[...] = jnp.where(sel, p / denom, 0.0)


@functools.partial(jax.jit, static_argnames=())
def kernel(inputs, W1, b1, W2, b2, routing_thresholds, usage_counts,
           specialization_scores):
    B, S, d = inputs.shape
    n = B * S
    x = inputs.reshape(n, d)
    w1t = W1.T.astype(jnp.bfloat16)
    w2t = W2.T.astype(jnp.bfloat16)
    grid = (n // TM,)

    out, maskf = pl.pallas_call(
        _router_block,
        grid=grid,
        in_specs=[
            pl.BlockSpec((TM, D), lambda i: (i, 0)),
            pl.BlockSpec((D, H), lambda i: (0, 0)),
            pl.BlockSpec((1, H), lambda i: (0, 0)),
            pl.BlockSpec((H, E), lambda i: (0, 0)),
            pl.BlockSpec((1, E), lambda i: (0, 0)),
            pl.BlockSpec((1, E), lambda i: (0, 0)),
            pl.BlockSpec((1, E), lambda i: (0, 0)),
            pl.BlockSpec((1, E), lambda i: (0, 0)),
        ],
        out_specs=[
            pl.BlockSpec((TM, E), lambda i: (i, 0)),
            pl.BlockSpec((1, E), lambda i: (0, 0)),
        ],
        out_shape=[
            jax.ShapeDtypeStruct((n, E), jnp.float32),
            jax.ShapeDtypeStruct((1, E), jnp.float32),
        ],
    )(x, w1t, b1.reshape(1, H), w2t, b2.reshape(1, E),
      routing_thresholds.reshape(1, E), usage_counts.reshape(1, E),
      specialization_scores.reshape(1, E))

    routing_weights = out.reshape(B, S, E)
    expert_mask = maskf.reshape(E) > 0.5
    return (routing_weights, expert_mask)


# speedup vs baseline: 1.3617x; 1.0279x over previous
import functools

import jax
import jax.numpy as jnp
from jax.experimental import pallas as pl
from jax.experimental.pallas import tpu as pltpu

D = 4096
H = D // 2
E = 8
TEMPERATURE = 1.0
MIN_USAGE_THRESHOLD = 0.01

TM = 1024


def _router_block(x_ref, w1t_ref, b1_ref, w2t_ref, b2_ref, thr_ref, uc_ref,
                  ss_ref, out_ref, mask_ref):
    xb = x_ref[...].astype(jnp.bfloat16)
    h = jax.lax.dot(xb, w1t_ref[...],
                    preferred_element_type=jnp.float32)
    h = jnp.maximum(h + b1_ref[...], 0.0)
    logits = jax.lax.dot(h.astype(jnp.bfloat16), w2t_ref[...],
                         preferred_element_type=jnp.float32)
    logits = (logits + b2_ref[...]) / TEMPERATURE

    uc = uc_ref[...]
    ur = uc / jnp.sum(uc)
    ss = ss_ref[...]
    sp = jnp.exp(ss - jnp.max(ss))
    sp = sp / jnp.sum(sp)
    maskf = jnp.where((ur > MIN_USAGE_THRESHOLD) & (sp > 0.05), 1.0, 0.0)
    mask_ref[...] = maskf

    logits = jnp.where(maskf > 0.5, logits, -jnp.inf)
    m = jnp.max(logits, axis=-1, keepdims=True)
    e = jnp.exp(logits - m)
    p = e / jnp.sum(e, axis=-1, keepdims=True)
    p = jnp.where(p > thr_ref[...], p, 0.0)

    iota = jax.lax.broadcasted_iota(jnp.int32, p.shape, 1)
    m1 = jnp.max(p, axis=-1, keepdims=True)
    i1 = jnp.min(jnp.where(p == m1, iota, E), axis=-1, keepdims=True)
    p2 = jnp.where(iota == i1, -1.0, p)
    m2 = jnp.max(p2, axis=-1, keepdims=True)
    i2 = jnp.min(jnp.where(p2 == m2, iota, E), axis=-1, keepdims=True)
    denom = m1 + m2 + 1e-9
    sel = (iota == i1) | (iota == i2)
    out_ref[...] = jnp.where(sel, p / denom, 0.0)


@functools.partial(jax.jit, static_argnames=())
def kernel(inputs, W1, b1, W2, b2, routing_thresholds, usage_counts,
           specialization_scores):
    B, S, d = inputs.shape
    n = B * S
    x = inputs.reshape(n, d)
    w1t = W1.T.astype(jnp.bfloat16)
    w2t = W2.T.astype(jnp.bfloat16)
    grid = (n // TM,)

    out, maskf = pl.pallas_call(
        _router_block,
        grid=grid,
        in_specs=[
            pl.BlockSpec((TM, D), lambda i: (i, 0)),
            pl.BlockSpec((D, H), lambda i: (0, 0)),
            pl.BlockSpec((1, H), lambda i: (0, 0)),
            pl.BlockSpec((H, E), lambda i: (0, 0)),
            pl.BlockSpec((1, E), lambda i: (0, 0)),
            pl.BlockSpec((1, E), lambda i: (0, 0)),
            pl.BlockSpec((1, E), lambda i: (0, 0)),
            pl.BlockSpec((1, E), lambda i: (0, 0)),
        ],
        out_specs=[
            pl.BlockSpec((TM, E), lambda i: (i, 0)),
            pl.BlockSpec((1, E), lambda i: (0, 0)),
        ],
        out_shape=[
            jax.ShapeDtypeStruct((n, E), jnp.float32),
            jax.ShapeDtypeStruct((1, E), jnp.float32),
        ],
        compiler_params=pltpu.CompilerParams(
            vmem_limit_bytes=64 * 1024 * 1024),
    )(x, w1t, b1.reshape(1, H), w2t, b2.reshape(1, E),
      routing_thresholds.reshape(1, E), usage_counts.reshape(1, E),
      specialization_scores.reshape(1, E))

    routing_weights = out.reshape(B, S, E)
    expert_mask = maskf.reshape(E) > 0.5
    return (routing_weights, expert_mask)


# speedup vs baseline: 1.3983x; 1.0269x over previous
import functools

import jax
import jax.numpy as jnp
from jax import lax
from jax.experimental import pallas as pl
from jax.experimental.pallas import tpu as pltpu
from jax.experimental.pallas import tpu_sc as plsc

D = 4096
H = D // 2
E = 8
TEMPERATURE = 1.0
MIN_USAGE_THRESHOLD = 0.01

TM = 1024

NC, NS = 2, 16
NW = NC * NS
VL = 16


def _mlp_block(x_ref, w1t_ref, b1_ref, w2t_ref, b2_ref, uc_ref, ss_ref,
               lt_ref, mask_ref):
    xb = x_ref[...].astype(jnp.bfloat16)
    h = jax.lax.dot(xb, w1t_ref[...],
                    preferred_element_type=jnp.float32)
    h = jnp.maximum(h + b1_ref[...], 0.0)
    logits = jax.lax.dot(h.astype(jnp.bfloat16), w2t_ref[...],
                         preferred_element_type=jnp.float32)
    logits = (logits + b2_ref[...]) / TEMPERATURE

    uc = uc_ref[...]
    ur = uc / jnp.sum(uc)
    ss = ss_ref[...]
    sp = jnp.exp(ss - jnp.max(ss))
    sp = sp / jnp.sum(sp)
    maskf = jnp.where((ur > MIN_USAGE_THRESHOLD) & (sp > 0.05), 1.0, 0.0)
    mask_ref[...] = maskf

    logits = jnp.where(maskf > 0.5, logits, -jnp.inf)
    lt_ref[...] = logits.T


def _mlp_logits(x, w1t, b1, w2t, b2, uc, ss, n):
    grid = (n // TM,)
    return pl.pallas_call(
        _mlp_block,
        grid=grid,
        in_specs=[
            pl.BlockSpec((TM, D), lambda i: (i, 0)),
            pl.BlockSpec((D, H), lambda i: (0, 0)),
            pl.BlockSpec((1, H), lambda i: (0, 0)),
            pl.BlockSpec((H, E), lambda i: (0, 0)),
            pl.BlockSpec((1, E), lambda i: (0, 0)),
            pl.BlockSpec((1, E), lambda i: (0, 0)),
            pl.BlockSpec((1, E), lambda i: (0, 0)),
        ],
        out_specs=[
            pl.BlockSpec((E, TM), lambda i: (0, i)),
            pl.BlockSpec((1, E), lambda i: (0, 0)),
        ],
        out_shape=[
            jax.ShapeDtypeStruct((E, n), jnp.float32),
            jax.ShapeDtypeStruct((1, E), jnp.float32),
        ],
        compiler_params=pltpu.CompilerParams(
            vmem_limit_bytes=64 * 1024 * 1024),
    )(x, w1t, b1.reshape(1, H), w2t, b2.reshape(1, E),
      uc.reshape(1, E), ss.reshape(1, E))


def _sc_tail(logits_t, thr_b, n):
    ch = n // NW
    mesh = plsc.VectorSubcoreMesh(core_axis_name="c", subcore_axis_name="s")

    @functools.partial(
        pl.kernel, mesh=mesh,
        out_type=jax.ShapeDtypeStruct((E, n), jnp.float32),
        scratch_types=[
            pltpu.VMEM((E, ch), jnp.float32),
            pltpu.VMEM((E, ch), jnp.float32),
            pltpu.VMEM((E, VL), jnp.float32),
            pltpu.SemaphoreType.DMA,
        ],
    )
    def tail(lg_hbm, thr_hbm, out_hbm, lg_v, out_v, thr_v, sem):
        wid = lax.axis_index("s") * NC + lax.axis_index("c")
        base = wid * ch
        pltpu.sync_copy(thr_hbm, thr_v)
        pltpu.async_copy(lg_hbm.at[:, pl.ds(base, ch)], lg_v, sem).wait()

        thr = [thr_v.at[e][...] for e in range(E)]

        @pl.loop(0, ch, step=VL)
        def _(c):
            l = [lg_v.at[e, pl.ds(c, VL)][...] for e in range(E)]
            m = l[0]
            for e in range(1, E):
                m = jnp.maximum(m, l[e])
            ex = [jnp.exp(l[e] - m) for e in range(E)]
            s = ex[0]
            for e in range(1, E):
                s = s + ex[e]
            p = [ex[e] / s for e in range(E)]
            p = [jnp.where(p[e] > thr[e], p[e], 0.0) for e in range(E)]

            m1 = p[0]
            for e in range(1, E):
                m1 = jnp.maximum(m1, p[e])
            i1 = jnp.where(p[0] == m1, 0.0, float(E))
            for e in range(1, E):
                i1 = jnp.minimum(i1, jnp.where(p[e] == m1, float(e), float(E)))
            p2 = [jnp.where(i1 == float(e), -1.0, p[e]) for e in range(E)]
            m2 = p2[0]
            for e in range(1, E):
                m2 = jnp.maximum(m2, p2[e])
            i2 = jnp.where(p2[0] == m2, 0.0, float(E))
            for e in range(1, E):
                i2 = jnp.minimum(i2, jnp.where(p2[e] == m2, float(e), float(E)))
            denom = m1 + m2 + 1e-9
            for e in range(E):
                sel = jnp.logical_or(i1 == float(e), i2 == float(e))
                out_v.at[e, pl.ds(c, VL)][...] = jnp.where(
                    sel, p[e] / denom, 0.0)

        pltpu.sync_copy(out_v, out_hbm.at[:, pl.ds(base, ch)])

    return tail(logits_t, thr_b)


@jax.jit
def kernel(inputs, W1, b1, W2, b2, routing_thresholds, usage_counts,
           specialization_scores):
    B, S, d = inputs.shape
    n = B * S
    x = inputs.reshape(n, d)
    w1t = W1.T.astype(jnp.bfloat16)
    w2t = W2.T.astype(jnp.bfloat16)

    logits_t, maskf = _mlp_logits(x, w1t, b1, w2t, b2, usage_counts,
                                  specialization_scores, n)
    thr_b = jnp.broadcast_to(routing_thresholds.reshape(E, 1), (E, VL))
    w_t = _sc_tail(logits_t, thr_b, n)

    routing_weights = w_t.T.reshape(B, S, E)
    expert_mask = maskf.reshape(E) > 0.5
    return (routing_weights, expert_mask)
